# Initial kernel scaffold; baseline (speedup 1.0000x reference)
import functools

import jax
import jax.numpy as jnp
from jax import lax
from jax.experimental import pallas as pl
from jax.experimental.pallas import tpu as pltpu
from jax.experimental.pallas import tpu_sc as plsc


def _build_wt(connt, num_neurons, total_bits, n_bits):
    info = plsc.get_sparse_core_info()
    nc, ns = info.num_cores, info.num_subcores
    nw = nc * ns
    rows_per = num_neurons // nw
    mesh = plsc.VectorSubcoreMesh(core_axis_name="c", subcore_axis_name="s")

    @functools.partial(
        pl.kernel,
        out_type=jax.ShapeDtypeStruct((num_neurons * total_bits,), jnp.float32),
        mesh=mesh,
        scratch_types=[
            pltpu.VMEM((n_bits, num_neurons), jnp.int32),
            pltpu.VMEM((rows_per * total_bits,), jnp.float32),
        ],
    )
    def wt_kernel(connt_hbm, wt_hbm, conn_v, wt_v):
        wid = lax.axis_index("s") * nc + lax.axis_index("c")
        n0 = wid * rows_per
        pltpu.sync_copy(connt_hbm, conn_v)
        nwords = rows_per * total_bits

        def zero_body(k, _):
            base = k * 128
            for u in range(8):
                wt_v[pl.ds(base + u * 16, 16)] = jnp.zeros((16,), jnp.float32)
            return 0

        lax.fori_loop(0, nwords // 128, zero_body, 0)
        iota = lax.iota(jnp.int32, 16)
        for k in range(rows_per // 16):
            local_n = k * 16 + iota
            for i in range(n_bits):
                w = float(1 << (n_bits - 1 - i))
                col = conn_v[i, pl.ds(n0 + k * 16, 16)]
                idx = local_n * total_bits + col
                plsc.addupdate_scatter(wt_v, [idx], jnp.full((16,), w, jnp.float32))
        pltpu.sync_copy(wt_v, wt_hbm.at[pl.ds(n0 * total_bits, nwords)])

    return wt_kernel(connt)


def _addresses_t(wt, input_bits, num_neurons, total_bits, batch):
    bb = 512

    def body(wt_ref, bits_ref, out_ref):
        bits = (bits_ref[...] != 0).astype(jnp.float32)
        acc = lax.dot_general(
            wt_ref[...], bits, (((1,), (1,)), ((), ())),
            preferred_element_type=jnp.float32,
        )
        out_ref[...] = acc.astype(jnp.int32)

    return pl.pallas_call(
        body,
        grid=(batch // bb,),
        in_specs=[
            pl.BlockSpec((num_neurons, total_bits), lambda j: (0, 0)),
            pl.BlockSpec((bb, total_bits), lambda j: (j, 0)),
        ],
        out_specs=pl.BlockSpec((num_neurons, bb), lambda j: (0, j)),
        out_shape=jax.ShapeDtypeStruct((num_neurons, batch), jnp.int32),
    )(wt, input_bits)


def _gather_t(memory, addrt, num_neurons, mem_size, batch):
    info = plsc.get_sparse_core_info()
    nc, ns = info.num_cores, info.num_subcores
    nw = nc * ns
    npt = num_neurons // nw
    mesh = plsc.VectorSubcoreMesh(core_axis_name="c", subcore_axis_name="s")

    @functools.partial(
        pl.kernel,
        out_type=jax.ShapeDtypeStruct((num_neurons, batch), jnp.float32),
        mesh=mesh,
        scratch_types=[
            pltpu.VMEM((mem_size,), jnp.float32),
            pltpu.VMEM((batch,), jnp.int32),
            pltpu.VMEM((batch,), jnp.float32),
        ],
    )
    def gather_kernel(mem_hbm, addrt_hbm, outt_hbm, mem_v, idx_v, out_v):
        wid = lax.axis_index("s") * nc + lax.axis_index("c")

        def neuron_body(t, _):
            nn = wid * npt + t
            pltpu.sync_copy(mem_hbm.at[nn], mem_v)
            pltpu.sync_copy(addrt_hbm.at[nn], idx_v)

            def gbody(k, _):
                base = k * 64
                for u in range(4):
                    ii = idx_v[pl.ds(base + u * 16, 16)]
                    out_v[pl.ds(base + u * 16, 16)] = plsc.load_gather(mem_v, [ii])
                return 0

            lax.fori_loop(0, batch // 64, gbody, 0)
            pltpu.sync_copy(out_v, outt_hbm.at[nn])
            return 0

        lax.fori_loop(0, npt, neuron_body, 0)

    return gather_kernel(memory, addrt)


def _transpose(outt, num_neurons, batch):
    bt = 512

    def body(in_ref, out_ref):
        out_ref[...] = in_ref[...].T

    return pl.pallas_call(
        body,
        grid=(batch // bt, num_neurons // bt),
        in_specs=[pl.BlockSpec((bt, bt), lambda i, j: (j, i))],
        out_specs=pl.BlockSpec((bt, bt), lambda i, j: (i, j)),
        out_shape=jax.ShapeDtypeStruct((batch, num_neurons), jnp.float32),
    )(outt)


def kernel(input_bits, connections, memory):
    batch, total_bits = input_bits.shape
    num_neurons, n_bits = connections.shape
    mem_size = memory.shape[1]

    connt = connections.T
    wt = _build_wt(connt, num_neurons, total_bits, n_bits)
    wt = wt.reshape(num_neurons, total_bits)
    addrt = _addresses_t(wt, input_bits, num_neurons, total_bits, batch)
    outt = _gather_t(memory, addrt, num_neurons, mem_size, batch)
    return _transpose(outt, num_neurons, batch)


# speedup vs baseline: 3.3997x; 3.3997x over previous
import functools

import jax
import jax.numpy as jnp
from jax import lax
from jax.experimental import pallas as pl
from jax.experimental.pallas import tpu as pltpu
from jax.experimental.pallas import tpu_sc as plsc


def _build_wt(connt, num_neurons, total_bits, n_bits):
    info = plsc.get_sparse_core_info()
    nc, ns = info.num_cores, info.num_subcores
    nw = nc * ns
    rows_per = num_neurons // nw
    mesh = plsc.VectorSubcoreMesh(core_axis_name="c", subcore_axis_name="s")

    @functools.partial(
        pl.kernel,
        out_type=jax.ShapeDtypeStruct((num_neurons * total_bits,), jnp.float32),
        mesh=mesh,
        compiler_params=pltpu.CompilerParams(needs_layout_passes=False),
        scratch_types=[
            pltpu.VMEM((n_bits, num_neurons), jnp.int32),
            pltpu.VMEM((rows_per * total_bits,), jnp.float32),
        ],
    )
    def wt_kernel(connt_hbm, wt_hbm, conn_v, wt_v):
        wid = lax.axis_index("s") * nc + lax.axis_index("c")
        n0 = wid * rows_per
        pltpu.sync_copy(connt_hbm, conn_v)
        nwords = rows_per * total_bits

        def zero_body(k, _):
            base = k * 128
            for u in range(8):
                wt_v[pl.ds(base + u * 16, 16)] = jnp.zeros((16,), jnp.float32)
            return 0

        lax.fori_loop(0, nwords // 128, zero_body, 0)
        iota = lax.iota(jnp.int32, 16)
        for k in range(rows_per // 16):
            local_n = k * 16 + iota
            for i in range(n_bits):
                w = float(1 << (n_bits - 1 - i))
                col = conn_v[i, pl.ds(n0 + k * 16, 16)]
                idx = local_n * total_bits + col
                plsc.addupdate_scatter(wt_v, [idx], jnp.full((16,), w, jnp.float32))
        pltpu.sync_copy(wt_v, wt_hbm.at[pl.ds(n0 * total_bits, nwords)])

    return wt_kernel(connt)


def _addresses_t(wt, input_bits, num_neurons, total_bits, batch):
    bb = 512

    def body(wt_ref, bits_ref, out_ref):
        bits = (bits_ref[...] != 0).astype(jnp.float32)
        acc = lax.dot_general(
            wt_ref[...], bits, (((1,), (1,)), ((), ())),
            precision=lax.Precision.HIGHEST,
            preferred_element_type=jnp.float32,
        )
        out_ref[...] = acc.astype(jnp.int32)

    return pl.pallas_call(
        body,
        grid=(batch // bb,),
        in_specs=[
            pl.BlockSpec((num_neurons, total_bits), lambda j: (0, 0)),
            pl.BlockSpec((bb, total_bits), lambda j: (j, 0)),
        ],
        out_specs=pl.BlockSpec((num_neurons, bb), lambda j: (0, j)),
        out_shape=jax.ShapeDtypeStruct((num_neurons, batch), jnp.int32),
    )(wt, input_bits)


def _gather_t(memory, addrt, num_neurons, mem_size, batch):
    info = plsc.get_sparse_core_info()
    nc, ns = info.num_cores, info.num_subcores
    nw = nc * ns
    npt = num_neurons // nw
    mesh = plsc.VectorSubcoreMesh(core_axis_name="c", subcore_axis_name="s")

    @functools.partial(
        pl.kernel,
        out_type=jax.ShapeDtypeStruct((num_neurons, batch), jnp.float32),
        mesh=mesh,
        compiler_params=pltpu.CompilerParams(needs_layout_passes=False),
        scratch_types=[
            pltpu.VMEM((mem_size,), jnp.float32),
            pltpu.VMEM((batch,), jnp.int32),
            pltpu.VMEM((batch,), jnp.float32),
        ],
    )
    def gather_kernel(mem_hbm, addrt_hbm, outt_hbm, mem_v, idx_v, out_v):
        wid = lax.axis_index("s") * nc + lax.axis_index("c")

        def neuron_body(t, _):
            nn = wid * npt + t
            pltpu.sync_copy(mem_hbm.at[nn], mem_v)
            pltpu.sync_copy(addrt_hbm.at[nn], idx_v)

            def gbody(k, _):
                base = k * 64
                for u in range(4):
                    ii = idx_v[pl.ds(base + u * 16, 16)]
                    out_v[pl.ds(base + u * 16, 16)] = plsc.load_gather(mem_v, [ii])
                return 0

            lax.fori_loop(0, batch // 64, gbody, 0)
            pltpu.sync_copy(out_v, outt_hbm.at[nn])
            return 0

        lax.fori_loop(0, npt, neuron_body, 0)

    return gather_kernel(memory, addrt)


def _transpose(outt, num_neurons, batch):
    bt = 512

    def body(in_ref, out_ref):
        out_ref[...] = in_ref[...].T

    return pl.pallas_call(
        body,
        grid=(batch // bt, num_neurons // bt),
        in_specs=[pl.BlockSpec((bt, bt), lambda i, j: (j, i))],
        out_specs=pl.BlockSpec((bt, bt), lambda i, j: (i, j)),
        out_shape=jax.ShapeDtypeStruct((batch, num_neurons), jnp.float32),
    )(outt)


def kernel(input_bits, connections, memory):
    batch, total_bits = input_bits.shape
    num_neurons, n_bits = connections.shape
    mem_size = memory.shape[1]

    connt = connections.T
    wt = _build_wt(connt, num_neurons, total_bits, n_bits)
    wt = wt.reshape(num_neurons, total_bits)
    addrt = _addresses_t(wt, input_bits, num_neurons, total_bits, batch)
    outt = _gather_t(memory, addrt, num_neurons, mem_size, batch)
    return _transpose(outt, num_neurons, batch)


# speedup vs baseline: 6.0423x; 1.7773x over previous
import functools

import jax
import jax.numpy as jnp
from jax import lax
from jax.experimental import pallas as pl
from jax.experimental.pallas import tpu as pltpu
from jax.experimental.pallas import tpu_sc as plsc


def _build_wt(connt, num_neurons, total_bits, n_bits):
    info = plsc.get_sparse_core_info()
    nc, ns = info.num_cores, info.num_subcores
    nw = nc * ns
    rows_per = num_neurons // nw
    mesh = plsc.VectorSubcoreMesh(core_axis_name="c", subcore_axis_name="s")

    @functools.partial(
        pl.kernel,
        out_type=jax.ShapeDtypeStruct((num_neurons * total_bits,), jnp.float32),
        mesh=mesh,
        compiler_params=pltpu.CompilerParams(needs_layout_passes=False),
        scratch_types=[
            pltpu.VMEM((n_bits, num_neurons), jnp.int32),
            pltpu.VMEM((rows_per * total_bits,), jnp.float32),
        ],
    )
    def wt_kernel(connt_hbm, wt_hbm, conn_v, wt_v):
        wid = lax.axis_index("s") * nc + lax.axis_index("c")
        n0 = wid * rows_per
        pltpu.sync_copy(connt_hbm, conn_v)
        nwords = rows_per * total_bits

        def zero_body(k, _):
            base = k * 128
            for u in range(8):
                wt_v[pl.ds(base + u * 16, 16)] = jnp.zeros((16,), jnp.float32)
            return 0

        lax.fori_loop(0, nwords // 128, zero_body, 0)
        iota = lax.iota(jnp.int32, 16)
        for k in range(rows_per // 16):
            local_n = k * 16 + iota
            for i in range(n_bits):
                w = float(1 << (n_bits - 1 - i))
                col = conn_v[i, pl.ds(n0 + k * 16, 16)]
                idx = local_n * total_bits + col
                plsc.addupdate_scatter(wt_v, [idx], jnp.full((16,), w, jnp.float32))
        pltpu.sync_copy(wt_v, wt_hbm.at[pl.ds(n0 * total_bits, nwords)])

    return wt_kernel(connt)


def _addresses_t(wt, input_bits, num_neurons, total_bits, batch):
    bb = 1024

    def body(wt_ref, bits_ref, out_ref):
        bits = (bits_ref[...] != 0).astype(jnp.bfloat16)
        wtv = wt_ref[...]
        hi = jnp.floor(wtv * (1.0 / 128.0))
        lo = wtv - hi * 128.0
        dn = (((1,), (1,)), ((), ()))
        acc_hi = lax.dot_general(
            hi.astype(jnp.bfloat16), bits, dn, preferred_element_type=jnp.float32
        )
        acc_lo = lax.dot_general(
            lo.astype(jnp.bfloat16), bits, dn, preferred_element_type=jnp.float32
        )
        out_ref[...] = (acc_hi * 128.0 + acc_lo).astype(jnp.int32)

    return pl.pallas_call(
        body,
        grid=(batch // bb,),
        in_specs=[
            pl.BlockSpec((num_neurons, total_bits), lambda j: (0, 0)),
            pl.BlockSpec((bb, total_bits), lambda j: (j, 0)),
        ],
        out_specs=pl.BlockSpec((num_neurons, bb), lambda j: (0, j)),
        out_shape=jax.ShapeDtypeStruct((num_neurons, batch), jnp.int32),
    )(wt, input_bits)


def _gather_t(memory, addrt, num_neurons, mem_size, batch):
    info = plsc.get_sparse_core_info()
    nc, ns = info.num_cores, info.num_subcores
    nw = nc * ns
    npt = num_neurons // nw
    mesh = plsc.VectorSubcoreMesh(core_axis_name="c", subcore_axis_name="s")

    @functools.partial(
        pl.kernel,
        out_type=jax.ShapeDtypeStruct((num_neurons, batch), jnp.float32),
        mesh=mesh,
        compiler_params=pltpu.CompilerParams(needs_layout_passes=False),
        scratch_types=[
            pltpu.VMEM((mem_size,), jnp.float32),
            pltpu.VMEM((mem_size,), jnp.float32),
            pltpu.VMEM((batch,), jnp.int32),
            pltpu.VMEM((batch,), jnp.int32),
            pltpu.VMEM((batch,), jnp.float32),
            pltpu.VMEM((batch,), jnp.float32),
            pltpu.SemaphoreType.DMA,
            pltpu.SemaphoreType.DMA,
            pltpu.SemaphoreType.DMA,
            pltpu.SemaphoreType.DMA,
            pltpu.SemaphoreType.DMA,
            pltpu.SemaphoreType.DMA,
        ],
    )
    def gather_kernel(mem_hbm, addrt_hbm, outt_hbm,
                      mem0, mem1, idx0, idx1, out0, out1,
                      sm0, sm1, si0, si1, so0, so1):
        wid = lax.axis_index("s") * nc + lax.axis_index("c")
        mem_b, idx_b, out_b = (mem0, mem1), (idx0, idx1), (out0, out1)
        sm_b, si_b, so_b = (sm0, sm1), (si0, si1), (so0, so1)

        def start_in(t, p):
            nn = wid * npt + t
            pltpu.make_async_copy(mem_hbm.at[nn], mem_b[p], sm_b[p]).start()
            pltpu.make_async_copy(addrt_hbm.at[nn], idx_b[p], si_b[p]).start()

        def wait_in(t, p):
            nn = wid * npt + t
            pltpu.make_async_copy(mem_hbm.at[nn], mem_b[p], sm_b[p]).wait()
            pltpu.make_async_copy(addrt_hbm.at[nn], idx_b[p], si_b[p]).wait()

        for t in range(npt):
            p = t & 1
            if t == 0:
                start_in(0, 0)
                start_in(1, 1)
            wait_in(t, p)
            if t >= 2:
                pltpu.make_async_copy(
                    out_b[p], outt_hbm.at[wid * npt + t - 2], so_b[p]
                ).wait()
            mv, iv, ov = mem_b[p], idx_b[p], out_b[p]

            def gbody(k, _, mv=mv, iv=iv, ov=ov):
                base = k * 64
                for u in range(4):
                    ii = iv[pl.ds(base + u * 16, 16)]
                    ov[pl.ds(base + u * 16, 16)] = plsc.load_gather(mv, [ii])
                return 0

            lax.fori_loop(0, batch // 64, gbody, 0)
            pltpu.make_async_copy(ov, outt_hbm.at[wid * npt + t], so_b[p]).start()
            if t + 2 < npt:
                start_in(t + 2, p)
        for t in range(max(npt - 2, 0), npt):
            p = t & 1
            pltpu.make_async_copy(
                out_b[p], outt_hbm.at[wid * npt + t], so_b[p]
            ).wait()

    return gather_kernel(memory, addrt)


def _transpose(outt, num_neurons, batch):
    bt = 512

    def body(in_ref, out_ref):
        out_ref[...] = in_ref[...].T

    return pl.pallas_call(
        body,
        grid=(batch // bt, num_neurons // bt),
        in_specs=[pl.BlockSpec((bt, bt), lambda i, j: (j, i))],
        out_specs=pl.BlockSpec((bt, bt), lambda i, j: (i, j)),
        out_shape=jax.ShapeDtypeStruct((batch, num_neurons), jnp.float32),
    )(outt)


def kernel(input_bits, connections, memory):
    batch, total_bits = input_bits.shape
    num_neurons, n_bits = connections.shape
    mem_size = memory.shape[1]

    connt = connections.T
    wt = _build_wt(connt, num_neurons, total_bits, n_bits)
    wt = wt.reshape(num_neurons, total_bits)
    addrt = _addresses_t(wt, input_bits, num_neurons, total_bits, batch)
    outt = _gather_t(memory, addrt, num_neurons, mem_size, batch)
    return _transpose(outt, num_neurons, batch)


# speedup vs baseline: 6.3682x; 1.0539x over previous
import functools

import jax
import jax.numpy as jnp
from jax import lax
from jax.experimental import pallas as pl
from jax.experimental.pallas import tpu as pltpu
from jax.experimental.pallas import tpu_sc as plsc


def _build_wt(connt, num_neurons, total_bits, n_bits):
    info = plsc.get_sparse_core_info()
    nc, ns = info.num_cores, info.num_subcores
    nw = nc * ns
    rows_per = num_neurons // nw
    mesh = plsc.VectorSubcoreMesh(core_axis_name="c", subcore_axis_name="s")

    @functools.partial(
        pl.kernel,
        out_type=jax.ShapeDtypeStruct((num_neurons * total_bits,), jnp.float32),
        mesh=mesh,
        compiler_params=pltpu.CompilerParams(needs_layout_passes=False),
        scratch_types=[
            pltpu.VMEM((n_bits, num_neurons), jnp.int32),
            pltpu.VMEM((rows_per * total_bits,), jnp.float32),
        ],
    )
    def wt_kernel(connt_hbm, wt_hbm, conn_v, wt_v):
        wid = lax.axis_index("s") * nc + lax.axis_index("c")
        n0 = wid * rows_per
        pltpu.sync_copy(connt_hbm, conn_v)
        nwords = rows_per * total_bits

        def zero_body(k, _):
            base = k * 128
            for u in range(8):
                wt_v[pl.ds(base + u * 16, 16)] = jnp.zeros((16,), jnp.float32)
            return 0

        lax.fori_loop(0, nwords // 128, zero_body, 0)
        iota = lax.iota(jnp.int32, 16)
        for k in range(rows_per // 16):
            local_n = k * 16 + iota
            for i in range(n_bits):
                w = float(1 << (n_bits - 1 - i))
                col = conn_v[i, pl.ds(n0 + k * 16, 16)]
                idx = local_n * total_bits + col
                plsc.addupdate_scatter(wt_v, [idx], jnp.full((16,), w, jnp.float32))
        pltpu.sync_copy(wt_v, wt_hbm.at[pl.ds(n0 * total_bits, nwords)])

    return wt_kernel(connt)


def _addresses_t(wt, input_bits, n_lo, n_chunk, total_bits, batch):
    bb = 1024
    ci = n_lo // n_chunk

    def body(wt_ref, bits_ref, out_ref):
        bits = (bits_ref[...] != 0).astype(jnp.bfloat16)
        wtv = wt_ref[...]
        hi = jnp.floor(wtv * (1.0 / 128.0))
        lo = wtv - hi * 128.0
        dn = (((1,), (1,)), ((), ()))
        acc_hi = lax.dot_general(
            hi.astype(jnp.bfloat16), bits, dn, preferred_element_type=jnp.float32
        )
        acc_lo = lax.dot_general(
            lo.astype(jnp.bfloat16), bits, dn, preferred_element_type=jnp.float32
        )
        out_ref[...] = (acc_hi * 128.0 + acc_lo).astype(jnp.int32)

    return pl.pallas_call(
        body,
        grid=(batch // bb,),
        in_specs=[
            pl.BlockSpec((n_chunk, total_bits), lambda j: (ci, 0)),
            pl.BlockSpec((bb, total_bits), lambda j: (j, 0)),
        ],
        out_specs=pl.BlockSpec((n_chunk, bb), lambda j: (0, j)),
        out_shape=jax.ShapeDtypeStruct((n_chunk, batch), jnp.int32),
    )(wt, input_bits)


def _gather_t(memory, addrt, n_lo, n_chunk, mem_size, batch):
    info = plsc.get_sparse_core_info()
    nc, ns = info.num_cores, info.num_subcores
    nw = nc * ns
    npt = n_chunk // nw
    mesh = plsc.VectorSubcoreMesh(core_axis_name="c", subcore_axis_name="s")

    @functools.partial(
        pl.kernel,
        out_type=jax.ShapeDtypeStruct((n_chunk, batch), jnp.float32),
        mesh=mesh,
        compiler_params=pltpu.CompilerParams(needs_layout_passes=False),
        scratch_types=[
            pltpu.VMEM((mem_size,), jnp.float32),
            pltpu.VMEM((mem_size,), jnp.float32),
            pltpu.VMEM((batch,), jnp.int32),
            pltpu.VMEM((batch,), jnp.int32),
            pltpu.VMEM((batch,), jnp.float32),
            pltpu.VMEM((batch,), jnp.float32),
            pltpu.SemaphoreType.DMA,
            pltpu.SemaphoreType.DMA,
            pltpu.SemaphoreType.DMA,
            pltpu.SemaphoreType.DMA,
            pltpu.SemaphoreType.DMA,
            pltpu.SemaphoreType.DMA,
        ],
    )
    def gather_kernel(mem_hbm, addrt_hbm, outt_hbm,
                      mem0, mem1, idx0, idx1, out0, out1,
                      sm0, sm1, si0, si1, so0, so1):
        wid = lax.axis_index("s") * nc + lax.axis_index("c")
        mem_b, idx_b, out_b = (mem0, mem1), (idx0, idx1), (out0, out1)
        sm_b, si_b, so_b = (sm0, sm1), (si0, si1), (so0, so1)

        def start_in(t, p):
            lt = wid * npt + t
            pltpu.make_async_copy(mem_hbm.at[n_lo + lt], mem_b[p], sm_b[p]).start()
            pltpu.make_async_copy(addrt_hbm.at[lt], idx_b[p], si_b[p]).start()

        def wait_in(t, p):
            lt = wid * npt + t
            pltpu.make_async_copy(mem_hbm.at[n_lo + lt], mem_b[p], sm_b[p]).wait()
            pltpu.make_async_copy(addrt_hbm.at[lt], idx_b[p], si_b[p]).wait()

        for t in range(npt):
            p = t & 1
            if t == 0:
                start_in(0, 0)
                start_in(1, 1)
            wait_in(t, p)
            if t >= 2:
                pltpu.make_async_copy(
                    out_b[p], outt_hbm.at[wid * npt + t - 2], so_b[p]
                ).wait()
            mv, iv, ov = mem_b[p], idx_b[p], out_b[p]

            def gbody(k, _, mv=mv, iv=iv, ov=ov):
                base = k * 64
                for u in range(4):
                    ii = iv[pl.ds(base + u * 16, 16)]
                    ov[pl.ds(base + u * 16, 16)] = plsc.load_gather(mv, [ii])
                return 0

            lax.fori_loop(0, batch // 64, gbody, 0)
            pltpu.make_async_copy(ov, outt_hbm.at[wid * npt + t], so_b[p]).start()
            if t + 2 < npt:
                start_in(t + 2, p)
        for t in range(max(npt - 2, 0), npt):
            p = t & 1
            pltpu.make_async_copy(
                out_b[p], outt_hbm.at[wid * npt + t], so_b[p]
            ).wait()

    return gather_kernel(memory, addrt)


def _transpose_into(outt_c, prev, n_lo, n_chunk, num_neurons, batch):
    bt = 512
    cj = n_lo // bt

    def body(*refs):
        in_ref, out_ref = refs[0], refs[-1]
        out_ref[...] = in_ref[...].T

    in_specs = [pl.BlockSpec((bt, bt), lambda i, j: (j, i))]
    args = [outt_c]
    aliases = {}
    if prev is not None:
        in_specs.append(pl.BlockSpec(memory_space=pltpu.MemorySpace.HBM))
        args.append(prev)
        aliases = {1: 0}
    return pl.pallas_call(
        body,
        grid=(batch // bt, n_chunk // bt),
        in_specs=in_specs,
        out_specs=pl.BlockSpec((bt, bt), lambda i, j: (i, cj + j)),
        out_shape=jax.ShapeDtypeStruct((batch, num_neurons), jnp.float32),
        input_output_aliases=aliases,
    )(*args)


def kernel(input_bits, connections, memory):
    batch, total_bits = input_bits.shape
    num_neurons, n_bits = connections.shape
    mem_size = memory.shape[1]
    n_chunks = 2
    n_chunk = num_neurons // n_chunks

    connt = connections.T
    wt = _build_wt(connt, num_neurons, total_bits, n_bits)
    wt = wt.reshape(num_neurons, total_bits)
    addrts = [
        _addresses_t(wt, input_bits, c * n_chunk, n_chunk, total_bits, batch)
        for c in range(n_chunks)
    ]
    outts = [
        _gather_t(memory, addrts[c], c * n_chunk, n_chunk, mem_size, batch)
        for c in range(n_chunks)
    ]
    out = None
    for c in range(n_chunks):
        out = _transpose_into(outts[c], out, c * n_chunk, n_chunk, num_neurons, batch)
    return out
